# baseline (speedup 1.0000x reference)
import functools

import jax
import jax.numpy as jnp
from jax import lax
from jax.experimental import pallas as pl
from jax.experimental.pallas import tpu as pltpu
from jax.experimental.pallas import tpu_sc as plsc

N = 10000
NP = 10240
E = 320000
CH = 128
NC, NS = 2, 16
NT = NC * NS
EP = 327680
NCHUNK = EP // CH
EPT = EP // NT
CPT = EPT // CH
HEPT = EP // NS
HCPT = HEPT // CH
RPT = NP // NS
TRASH = 10200
KB = 2
KH = 8

_MESH = dict(core_axis_name="c", subcore_axis_name="s")


def _sc_pass(F):
    @functools.partial(
        pl.kernel,
        out_type=jax.ShapeDtypeStruct((NC * NP, F), jnp.float32),
        mesh=plsc.VectorSubcoreMesh(**_MESH),
        scratch_types=[
            pltpu.VMEM_SHARED((NP, F), jnp.float32),
            pltpu.VMEM((1, 2, CH), jnp.int32),
            pltpu.VMEM((1, 2, CH), jnp.int32),
            pltpu.VMEM((CH, F), jnp.float32),
            pltpu.VMEM((CH, F), jnp.float32),
            pltpu.SemaphoreType.DMA,
            pltpu.SemaphoreType.DMA,
        ],
    )
    def k(src, packed, zrows, out, acc, iva, ivb, rowsa, rowsb, sema, semb):
        c = lax.axis_index("c")
        s = lax.axis_index("s")
        w = c * NS + s
        pltpu.sync_copy(zrows, rowsa)
        for kk in range(RPT // CH):
            pltpu.sync_copy(rowsa, acc.at[pl.ds(s * RPT + kk * CH, CH)])
        plsc.subcore_barrier()

        @pl.loop(0, CPT // 2)
        def _(r):
            cb = w * CPT + 2 * r
            pltpu.sync_copy(packed.at[pl.ds(cb, 1)], iva)
            pltpu.sync_copy(packed.at[pl.ds(cb + 1, 1)], ivb)
            da = pltpu.async_copy(src.at[iva.at[0, 0]], rowsa, sema)
            db = pltpu.async_copy(src.at[ivb.at[0, 0]], rowsb, semb)
            da.wait()
            pltpu.sync_copy(rowsa, acc.at[iva.at[0, 1]], add=True)
            db.wait()
            pltpu.sync_copy(rowsb, acc.at[ivb.at[0, 1]], add=True)

        plsc.subcore_barrier()
        pltpu.sync_copy(acc.at[pl.ds(s * RPT, RPT)],
                        out.at[pl.ds(c * NP + s * RPT, RPT)])

    return k


@functools.partial(
    pl.kernel,
    out_type=jax.ShapeDtypeStruct((NC * NP, 128), jnp.float32),
    mesh=plsc.VectorSubcoreMesh(**_MESH),
    scratch_types=(
        [pltpu.VMEM_SHARED((NP, 128), jnp.float32),
         pltpu.VMEM((KH, CH), jnp.int32),
         pltpu.VMEM((CH, 128), jnp.float32)]
        + [pltpu.SemaphoreType.DMA for _ in range(KH)]
    ),
)
def _sc_hist(nhid, zrows, out, acc, iv, ones_v, *sems):
    c = lax.axis_index("c")
    s = lax.axis_index("s")
    pltpu.sync_copy(zrows, ones_v)
    for kk in range(RPT // CH):
        pltpu.sync_copy(ones_v, acc.at[pl.ds(s * RPT + kk * CH, CH)])
    e0 = jnp.where(lax.iota(jnp.int32, 16) == 0, 1.0, 0.0)

    @pl.loop(0, CH)
    def _(i):
        ones_v[i, pl.ds(0, 16)] = e0

    plsc.subcore_barrier()

    @pl.loop(0, HCPT // KH)
    def _(r):
        cb = c * NCHUNK + s * HCPT + r * KH
        pltpu.sync_copy(nhid.at[pl.ds(cb, KH)], iv)
        ds = [pltpu.async_copy(ones_v, acc.at[iv.at[b]], sems[b], add=True)
              for b in range(KH)]
        for d in ds:
            d.wait()

    plsc.subcore_barrier()
    pltpu.sync_copy(acc.at[pl.ds(s * RPT, RPT)],
                    out.at[pl.ds(c * NP + s * RPT, RPT)])


_BR = 1024


def _mm1(x, W1):
    def body(xr, wr, o):
        o[...] = jnp.dot(xr[...], wr[...], preferred_element_type=jnp.float32)

    return pl.pallas_call(
        body,
        grid=(NP // _BR,),
        in_specs=[
            pl.BlockSpec((_BR, 128), lambda i: (i, 0)),
            pl.BlockSpec((128, 128), lambda i: (0, 0)),
        ],
        out_specs=pl.BlockSpec((_BR, 128), lambda i: (i, 0)),
        out_shape=jax.ShapeDtypeStruct((NP, 128), jnp.float32),
    )(x, W1)


def _scale(p, hist, F, slab):
    nb = NP // _BR

    def body(p0, p1, hr, o):
        den = jnp.maximum(hr[...][:, 0:1], 1.0)
        o[...] = (p0[...] + p1[...]) / den

    return pl.pallas_call(
        body,
        grid=(nb,),
        in_specs=[
            pl.BlockSpec((_BR, F), lambda i: (i, 0)),
            pl.BlockSpec((_BR, F), lambda i: (i + nb, 0)),
            pl.BlockSpec((_BR, 128), lambda i, _s=slab * nb: (i + _s, 0)),
        ],
        out_specs=pl.BlockSpec((_BR, F), lambda i: (i, 0)),
        out_shape=jax.ShapeDtypeStruct((NP, F), jnp.float32),
    )(p, p, hist)


def _s2(q, hist, b1, W2p):
    nb = NP // _BR

    def body(q0, q1, hr, br, wr, o):
        den = jnp.maximum(hr[...][:, 0:1], 1.0)
        h = jnp.maximum((q0[...] + q1[...]) / den + br[...], 0.0)
        o[...] = jnp.dot(h, wr[...], preferred_element_type=jnp.float32)

    return pl.pallas_call(
        body,
        grid=(nb,),
        in_specs=[
            pl.BlockSpec((_BR, 128), lambda i: (i, 0)),
            pl.BlockSpec((_BR, 128), lambda i: (i + nb, 0)),
            pl.BlockSpec((_BR, 128), lambda i: (i, 0)),
            pl.BlockSpec((1, 128), lambda i: (0, 0)),
            pl.BlockSpec((128, 128), lambda i: (0, 0)),
        ],
        out_specs=pl.BlockSpec((_BR, 128), lambda i: (i, 0)),
        out_shape=jax.ShapeDtypeStruct((NP, 128), jnp.float32),
    )(q, q, hist, b1, W2p)


def _s4(r, hist, b2):
    nb = NP // _BR

    def body(r0, r1, hr, br, o):
        den = jnp.maximum(hr[...][:, 0:1], 1.0)
        o[...] = (r0[...] + r1[...])[:, :64] / den + br[...]

    return pl.pallas_call(
        body,
        grid=(nb,),
        in_specs=[
            pl.BlockSpec((_BR, 128), lambda i: (i, 0)),
            pl.BlockSpec((_BR, 128), lambda i: (i + nb, 0)),
            pl.BlockSpec((_BR, 128), lambda i: (i, 0)),
            pl.BlockSpec((1, 64), lambda i: (0, 0)),
        ],
        out_specs=pl.BlockSpec((_BR, 64), lambda i: (i, 0)),
        out_shape=jax.ShapeDtypeStruct((NP, 64), jnp.float32),
    )(r, r, hist, b2)


_pass128 = _sc_pass(128)


def kernel(x, edge_index, W1, b1, W2, b2):
    nid = edge_index[0]
    hid = edge_index[1]
    pad = jnp.full((EP - E,), TRASH, dtype=jnp.int32)
    nid_p = jnp.concatenate([nid, pad]).reshape(NCHUNK, 1, CH)
    hid_p = jnp.concatenate([hid, pad]).reshape(NCHUNK, 1, CH)
    junk = jnp.full((1, 2, CH), TRASH, dtype=jnp.int32)
    pk_ns = jnp.concatenate(
        [jnp.concatenate([nid_p, hid_p], axis=1), junk])
    pk_sn = jnp.concatenate(
        [jnp.concatenate([hid_p, nid_p], axis=1), junk])
    nhid = jnp.concatenate([nid_p, hid_p]).reshape(2 * NCHUNK, CH)
    x_pad = jnp.zeros((NP, 128), jnp.float32).at[:N].set(x)
    z128 = jnp.zeros((CH, 128), jnp.float32)
    W2p = jnp.zeros((128, 128), jnp.float32).at[:, :64].set(W2)

    hist = _sc_hist(nhid, z128)
    xt1 = _mm1(x_pad, W1)
    p = _pass128(xt1, pk_ns, z128)
    he = _scale(p, hist, 128, 1)
    q = _pass128(he, pk_sn, z128)
    xt2 = _s2(q, hist, b1.reshape(1, -1), W2p)
    u = _pass128(xt2, pk_ns, z128)
    he2 = _scale(u, hist, 128, 1)
    r = _pass128(he2, pk_sn, z128)
    out = _s4(r, hist, b2.reshape(1, -1))
    return out[:N]


# speedup vs baseline: 1.0172x; 1.0172x over previous
import functools

import jax
import jax.numpy as jnp
from jax import lax
from jax.experimental import pallas as pl
from jax.experimental.pallas import tpu as pltpu
from jax.experimental.pallas import tpu_sc as plsc

N = 10000
NP = 10240
E = 320000
CH = 128
NC, NS = 2, 16
NT = NC * NS
EP = 327680
NCHUNK = EP // CH
EPT = EP // NT
CPT = EPT // CH
HEPT = EP // NS
HCPT = HEPT // CH
RPT = NP // NS
TRASH = 10200
KB = 8
KH = 8

_MESH = dict(core_axis_name="c", subcore_axis_name="s")


def _sc_pass(F):
    @functools.partial(
        pl.kernel,
        out_type=jax.ShapeDtypeStruct((NC * NP, F), jnp.float32),
        mesh=plsc.VectorSubcoreMesh(**_MESH),
        scratch_types=[
            pltpu.VMEM_SHARED((NP, F), jnp.float32),
            pltpu.VMEM((KB, 2, CH), jnp.int32),
            pltpu.VMEM((CH, F), jnp.float32),
            pltpu.SemaphoreType.DMA,
        ],
    )
    def k(src, packed, zrows, out, acc, iv, rows, sem):
        c = lax.axis_index("c")
        s = lax.axis_index("s")
        w = c * NS + s
        pltpu.sync_copy(zrows, rows)
        for kk in range(RPT // CH):
            pltpu.sync_copy(rows, acc.at[pl.ds(s * RPT + kk * CH, CH)])
        plsc.subcore_barrier()

        @pl.loop(0, CPT // KB)
        def _(r):
            cb = w * CPT + r * KB
            pltpu.sync_copy(packed.at[pl.ds(cb, KB)], iv)
            for b in range(KB):
                pltpu.async_copy(src.at[iv.at[b, 0]], rows, sem).wait()
                pltpu.sync_copy(rows, acc.at[iv.at[b, 1]], add=True)

        plsc.subcore_barrier()
        pltpu.sync_copy(acc.at[pl.ds(s * RPT, RPT)],
                        out.at[pl.ds(c * NP + s * RPT, RPT)])

    return k


@functools.partial(
    pl.kernel,
    out_type=jax.ShapeDtypeStruct((NC * NP, 128), jnp.float32),
    mesh=plsc.VectorSubcoreMesh(**_MESH),
    scratch_types=(
        [pltpu.VMEM_SHARED((NP, 128), jnp.float32),
         pltpu.VMEM((KH, CH), jnp.int32),
         pltpu.VMEM((CH, 128), jnp.float32)]
    ),
)
def _sc_hist(nhid, zrows, out, acc, iv, ones_v):
    c = lax.axis_index("c")
    s = lax.axis_index("s")
    pltpu.sync_copy(zrows, ones_v)
    for kk in range(RPT // CH):
        pltpu.sync_copy(ones_v, acc.at[pl.ds(s * RPT + kk * CH, CH)])
    e0 = jnp.where(lax.iota(jnp.int32, 16) == 0, 1.0, 0.0)

    @pl.loop(0, CH)
    def _(i):
        ones_v[i, pl.ds(0, 16)] = e0

    plsc.subcore_barrier()

    @pl.loop(0, HCPT // KH)
    def _(r):
        cb = c * NCHUNK + s * HCPT + r * KH
        pltpu.sync_copy(nhid.at[pl.ds(cb, KH)], iv)
        for b in range(KH):
            pltpu.sync_copy(ones_v, acc.at[iv.at[b]], add=True)

    plsc.subcore_barrier()
    pltpu.sync_copy(acc.at[pl.ds(s * RPT, RPT)],
                    out.at[pl.ds(c * NP + s * RPT, RPT)])


_BR = 1024


def _mm1(x, W1):
    def body(xr, wr, o):
        o[...] = jnp.dot(xr[...], wr[...], preferred_element_type=jnp.float32)

    return pl.pallas_call(
        body,
        grid=(NP // _BR,),
        in_specs=[
            pl.BlockSpec((_BR, 128), lambda i: (i, 0)),
            pl.BlockSpec((128, 128), lambda i: (0, 0)),
        ],
        out_specs=pl.BlockSpec((_BR, 128), lambda i: (i, 0)),
        out_shape=jax.ShapeDtypeStruct((NP, 128), jnp.float32),
    )(x, W1)


def _scale(p, hist, F, slab):
    nb = NP // _BR

    def body(p0, p1, hr, o):
        den = jnp.maximum(hr[...][:, 0:1], 1.0)
        o[...] = (p0[...] + p1[...]) / den

    return pl.pallas_call(
        body,
        grid=(nb,),
        in_specs=[
            pl.BlockSpec((_BR, F), lambda i: (i, 0)),
            pl.BlockSpec((_BR, F), lambda i: (i + nb, 0)),
            pl.BlockSpec((_BR, 128), lambda i, _s=slab * nb: (i + _s, 0)),
        ],
        out_specs=pl.BlockSpec((_BR, F), lambda i: (i, 0)),
        out_shape=jax.ShapeDtypeStruct((NP, F), jnp.float32),
    )(p, p, hist)


def _s2(q, hist, b1, W2p):
    nb = NP // _BR

    def body(q0, q1, hr, br, wr, o):
        den = jnp.maximum(hr[...][:, 0:1], 1.0)
        h = jnp.maximum((q0[...] + q1[...]) / den + br[...], 0.0)
        o[...] = jnp.dot(h, wr[...], preferred_element_type=jnp.float32)

    return pl.pallas_call(
        body,
        grid=(nb,),
        in_specs=[
            pl.BlockSpec((_BR, 128), lambda i: (i, 0)),
            pl.BlockSpec((_BR, 128), lambda i: (i + nb, 0)),
            pl.BlockSpec((_BR, 128), lambda i: (i, 0)),
            pl.BlockSpec((1, 128), lambda i: (0, 0)),
            pl.BlockSpec((128, 128), lambda i: (0, 0)),
        ],
        out_specs=pl.BlockSpec((_BR, 128), lambda i: (i, 0)),
        out_shape=jax.ShapeDtypeStruct((NP, 128), jnp.float32),
    )(q, q, hist, b1, W2p)


def _s4(r, hist, b2):
    nb = NP // _BR

    def body(r0, r1, hr, br, o):
        den = jnp.maximum(hr[...][:, 0:1], 1.0)
        o[...] = (r0[...] + r1[...])[:, :64] / den + br[...]

    return pl.pallas_call(
        body,
        grid=(nb,),
        in_specs=[
            pl.BlockSpec((_BR, 128), lambda i: (i, 0)),
            pl.BlockSpec((_BR, 128), lambda i: (i + nb, 0)),
            pl.BlockSpec((_BR, 128), lambda i: (i, 0)),
            pl.BlockSpec((1, 64), lambda i: (0, 0)),
        ],
        out_specs=pl.BlockSpec((_BR, 64), lambda i: (i, 0)),
        out_shape=jax.ShapeDtypeStruct((NP, 64), jnp.float32),
    )(r, r, hist, b2)


_pass128 = _sc_pass(128)


def kernel(x, edge_index, W1, b1, W2, b2):
    nid = edge_index[0]
    hid = edge_index[1]
    pad = jnp.full((EP - E,), TRASH, dtype=jnp.int32)
    nid_p = jnp.concatenate([nid, pad]).reshape(NCHUNK, 1, CH)
    hid_p = jnp.concatenate([hid, pad]).reshape(NCHUNK, 1, CH)
    junk = jnp.full((1, 2, CH), TRASH, dtype=jnp.int32)
    pk_ns = jnp.concatenate(
        [jnp.concatenate([nid_p, hid_p], axis=1), junk])
    pk_sn = jnp.concatenate(
        [jnp.concatenate([hid_p, nid_p], axis=1), junk])
    nhid = jnp.concatenate([nid_p, hid_p]).reshape(2 * NCHUNK, CH)
    x_pad = jnp.zeros((NP, 128), jnp.float32).at[:N].set(x)
    z128 = jnp.zeros((CH, 128), jnp.float32)
    W2p = jnp.zeros((128, 128), jnp.float32).at[:, :64].set(W2)

    hist = _sc_hist(nhid, z128)
    xt1 = _mm1(x_pad, W1)
    p = _pass128(xt1, pk_ns, z128)
    he = _scale(p, hist, 128, 1)
    q = _pass128(he, pk_sn, z128)
    xt2 = _s2(q, hist, b1.reshape(1, -1), W2p)
    u = _pass128(xt2, pk_ns, z128)
    he2 = _scale(u, hist, 128, 1)
    r = _pass128(he2, pk_sn, z128)
    out = _s4(r, hist, b2.reshape(1, -1))
    return out[:N]


# speedup vs baseline: 1.0909x; 1.0724x over previous
import functools

import jax
import jax.numpy as jnp
from jax import lax
from jax.experimental import pallas as pl
from jax.experimental.pallas import tpu as pltpu
from jax.experimental.pallas import tpu_sc as plsc

N = 10000
NP = 10240
E = 320000
CH = 128
NC, NS = 2, 16
NT = NC * NS
EP = 327680
EPT = EP // NT
SUP = 256
SPT = EPT // SUP
HEPT = EP // NS
HCPT = HEPT // CH
RPT = NP // NS
TRASH = 10200

_MESH = dict(core_axis_name="c", subcore_axis_name="s")


def _sc_pass(F):
    @functools.partial(
        pl.kernel,
        out_type=jax.ShapeDtypeStruct((NC * NP, F), jnp.float32),
        mesh=plsc.VectorSubcoreMesh(**_MESH),
        scratch_types=[
            pltpu.VMEM_SHARED((NP, F), jnp.float32),
            pltpu.VMEM((SUP,), jnp.int32),
            pltpu.VMEM((SUP,), jnp.int32),
            pltpu.VMEM((SUP, F), jnp.float32),
            pltpu.SemaphoreType.DMA,
        ],
    )
    def k(src, gidx, sidx, zrows, out, acc, gi, si, rows, sem):
        c = lax.axis_index("c")
        s = lax.axis_index("s")
        w = c * NS + s
        pltpu.sync_copy(zrows, rows.at[pl.ds(0, CH)])
        for kk in range(RPT // CH):
            pltpu.sync_copy(rows.at[pl.ds(0, CH)],
                            acc.at[pl.ds(s * RPT + kk * CH, CH)])
        plsc.subcore_barrier()

        @pl.loop(0, SPT)
        def _(ci):
            off = w * EPT + ci * SUP
            pltpu.sync_copy(gidx.at[pl.ds(off, SUP)], gi)
            pltpu.sync_copy(sidx.at[pl.ds(off, SUP)], si)
            pltpu.async_copy(src.at[gi], rows, sem).wait()
            pltpu.sync_copy(rows, acc.at[si], add=True)

        plsc.subcore_barrier()
        pltpu.sync_copy(acc.at[pl.ds(s * RPT, RPT)],
                        out.at[pl.ds(c * NP + s * RPT, RPT)])

    return k


@functools.partial(
    pl.kernel,
    out_type=jax.ShapeDtypeStruct((NC * NP, 128), jnp.float32),
    mesh=plsc.VectorSubcoreMesh(**_MESH),
    scratch_types=[
        pltpu.VMEM_SHARED((NP, 128), jnp.float32),
        pltpu.VMEM((CH,), jnp.int32),
        pltpu.VMEM((CH, 128), jnp.float32),
        pltpu.VMEM((CH, 128), jnp.float32),
    ],
)
def _sc_hist(nhid, zrows, out, acc, ii, ones_v, z_v):
    c = lax.axis_index("c")
    s = lax.axis_index("s")
    pltpu.sync_copy(zrows, z_v)
    pltpu.sync_copy(zrows, ones_v)
    e0 = jnp.where(lax.iota(jnp.int32, 16) == 0, 1.0, 0.0)

    @pl.loop(0, CH)
    def _(i):
        ones_v[i, pl.ds(0, 16)] = e0

    for kk in range(RPT // CH):
        pltpu.sync_copy(z_v, acc.at[pl.ds(s * RPT + kk * CH, CH)])
    plsc.subcore_barrier()

    @pl.loop(0, HCPT)
    def _(ci):
        off = c * EP + s * HEPT + ci * CH
        pltpu.sync_copy(nhid.at[pl.ds(off, CH)], ii)
        pltpu.sync_copy(ones_v, acc.at[ii], add=True)

    plsc.subcore_barrier()
    pltpu.sync_copy(acc.at[pl.ds(s * RPT, RPT)],
                    out.at[pl.ds(c * NP + s * RPT, RPT)])


_BR = 1024


def _mm1(x, W1):
    def body(xr, wr, o):
        o[...] = jnp.dot(xr[...], wr[...], preferred_element_type=jnp.float32)

    return pl.pallas_call(
        body,
        grid=(NP // _BR,),
        in_specs=[
            pl.BlockSpec((_BR, 128), lambda i: (i, 0)),
            pl.BlockSpec((128, 128), lambda i: (0, 0)),
        ],
        out_specs=pl.BlockSpec((_BR, 128), lambda i: (i, 0)),
        out_shape=jax.ShapeDtypeStruct((NP, 128), jnp.float32),
    )(x, W1)


def _scale(p, hist, F, slab):
    nb = NP // _BR

    def body(p0, p1, hr, o):
        den = jnp.maximum(hr[...][:, 0:1], 1.0)
        o[...] = (p0[...] + p1[...]) / den

    return pl.pallas_call(
        body,
        grid=(nb,),
        in_specs=[
            pl.BlockSpec((_BR, F), lambda i: (i, 0)),
            pl.BlockSpec((_BR, F), lambda i: (i + nb, 0)),
            pl.BlockSpec((_BR, 128), lambda i, _s=slab * nb: (i + _s, 0)),
        ],
        out_specs=pl.BlockSpec((_BR, F), lambda i: (i, 0)),
        out_shape=jax.ShapeDtypeStruct((NP, F), jnp.float32),
    )(p, p, hist)


def _s2(q, hist, b1, W2p):
    nb = NP // _BR

    def body(q0, q1, hr, br, wr, o):
        den = jnp.maximum(hr[...][:, 0:1], 1.0)
        h = jnp.maximum((q0[...] + q1[...]) / den + br[...], 0.0)
        o[...] = jnp.dot(h, wr[...], preferred_element_type=jnp.float32)

    return pl.pallas_call(
        body,
        grid=(nb,),
        in_specs=[
            pl.BlockSpec((_BR, 128), lambda i: (i, 0)),
            pl.BlockSpec((_BR, 128), lambda i: (i + nb, 0)),
            pl.BlockSpec((_BR, 128), lambda i: (i, 0)),
            pl.BlockSpec((1, 128), lambda i: (0, 0)),
            pl.BlockSpec((128, 128), lambda i: (0, 0)),
        ],
        out_specs=pl.BlockSpec((_BR, 128), lambda i: (i, 0)),
        out_shape=jax.ShapeDtypeStruct((NP, 128), jnp.float32),
    )(q, q, hist, b1, W2p)


def _s4(r, hist, b2):
    nb = NP // _BR

    def body(r0, r1, hr, br, o):
        den = jnp.maximum(hr[...][:, 0:1], 1.0)
        o[...] = (r0[...] + r1[...])[:, :64] / den + br[...]

    return pl.pallas_call(
        body,
        grid=(nb,),
        in_specs=[
            pl.BlockSpec((_BR, 128), lambda i: (i, 0)),
            pl.BlockSpec((_BR, 128), lambda i: (i + nb, 0)),
            pl.BlockSpec((_BR, 128), lambda i: (i, 0)),
            pl.BlockSpec((1, 64), lambda i: (0, 0)),
        ],
        out_specs=pl.BlockSpec((_BR, 64), lambda i: (i, 0)),
        out_shape=jax.ShapeDtypeStruct((NP, 64), jnp.float32),
    )(r, r, hist, b2)


_pass128 = _sc_pass(128)


def kernel(x, edge_index, W1, b1, W2, b2):
    nid = edge_index[0]
    hid = edge_index[1]
    pad = jnp.full((EP - E,), TRASH, dtype=jnp.int32)
    nid_p = jnp.concatenate([nid, pad])
    hid_p = jnp.concatenate([hid, pad])
    x_pad = jnp.zeros((NP, 128), jnp.float32).at[:N].set(x)
    z128 = jnp.zeros((CH, 128), jnp.float32)
    W2p = jnp.zeros((128, 128), jnp.float32).at[:, :64].set(W2)

    hist = _sc_hist(jnp.concatenate([nid_p, hid_p]), z128)
    xt1 = _mm1(x_pad, W1)
    p = _pass128(xt1, nid_p, hid_p, z128)
    he = _scale(p, hist, 128, 1)
    q = _pass128(he, hid_p, nid_p, z128)
    xt2 = _s2(q, hist, b1.reshape(1, -1), W2p)
    u = _pass128(xt2, nid_p, hid_p, z128)
    he2 = _scale(u, hist, 128, 1)
    r = _pass128(he2, hid_p, nid_p, z128)
    out = _s4(r, hist, b2.reshape(1, -1))
    return out[:N]


# speedup vs baseline: 1.2175x; 1.1161x over previous
import functools

import jax
import jax.numpy as jnp
from jax import lax
from jax.experimental import pallas as pl
from jax.experimental.pallas import tpu as pltpu
from jax.experimental.pallas import tpu_sc as plsc

N = 10000
NP = 10240
E = 320000
CH = 128
NC, NS = 2, 16
NT = NC * NS
EP = 323584
EPT = EP // NT
CPT = EPT // CH
HEPT = EP // NS
HCPT = HEPT // CH
RPT = NP // NS
TRASH = 10200

_MESH = dict(core_axis_name="c", subcore_axis_name="s")


def _sc_pass(F):
    @functools.partial(
        pl.kernel,
        out_type=jax.ShapeDtypeStruct((NC * NP, F), jnp.float32),
        mesh=plsc.VectorSubcoreMesh(**_MESH),
        scratch_types=[
            pltpu.VMEM_SHARED((NP, F), jnp.float32),
            pltpu.VMEM((CH,), jnp.int32),
            pltpu.VMEM((CH,), jnp.int32),
            pltpu.VMEM((CH, F), jnp.float32),
            pltpu.SemaphoreType.DMA,
        ],
    )
    def k(src, gidx, sidx, zrows, out, acc, gi, si, rows, sem):
        c = lax.axis_index("c")
        s = lax.axis_index("s")
        w = c * NS + s
        pltpu.sync_copy(zrows, rows)
        for kk in range(RPT // CH):
            pltpu.sync_copy(rows, acc.at[pl.ds(s * RPT + kk * CH, CH)])
        plsc.subcore_barrier()

        @pl.loop(0, CPT)
        def _(ci):
            off = w * EPT + ci * CH
            pltpu.sync_copy(gidx.at[pl.ds(off, CH)], gi)
            pltpu.sync_copy(sidx.at[pl.ds(off, CH)], si)
            pltpu.async_copy(src.at[gi], rows, sem).wait()
            pltpu.sync_copy(rows, acc.at[si], add=True)

        plsc.subcore_barrier()
        pltpu.sync_copy(acc.at[pl.ds(s * RPT, RPT)],
                        out.at[pl.ds(c * NP + s * RPT, RPT)])

    return k


@functools.partial(
    pl.kernel,
    out_type=jax.ShapeDtypeStruct((NC * NP, 128), jnp.float32),
    mesh=plsc.VectorSubcoreMesh(**_MESH),
    scratch_types=[
        pltpu.VMEM_SHARED((NP, 128), jnp.float32),
        pltpu.VMEM((CH,), jnp.int32),
        pltpu.VMEM((CH, 128), jnp.float32),
        pltpu.VMEM((CH, 128), jnp.float32),
    ],
)
def _sc_hist(nhid, zrows, out, acc, ii, ones_v, z_v):
    c = lax.axis_index("c")
    s = lax.axis_index("s")
    pltpu.sync_copy(zrows, z_v)
    pltpu.sync_copy(zrows, ones_v)
    e0 = jnp.where(lax.iota(jnp.int32, 16) == 0, 1.0, 0.0)

    @pl.loop(0, CH)
    def _(i):
        ones_v[i, pl.ds(0, 16)] = e0

    for kk in range(RPT // CH):
        pltpu.sync_copy(z_v, acc.at[pl.ds(s * RPT + kk * CH, CH)])
    plsc.subcore_barrier()

    @pl.loop(0, HCPT)
    def _(ci):
        off = c * EP + s * HEPT + ci * CH
        pltpu.sync_copy(nhid.at[pl.ds(off, CH)], ii)
        pltpu.sync_copy(ones_v, acc.at[ii], add=True)

    plsc.subcore_barrier()
    pltpu.sync_copy(acc.at[pl.ds(s * RPT, RPT)],
                    out.at[pl.ds(c * NP + s * RPT, RPT)])


_BR = 1024


def _mm1(x, W1):
    def body(xr, wr, o):
        o[...] = jnp.dot(xr[...], wr[...], preferred_element_type=jnp.float32)

    return pl.pallas_call(
        body,
        grid=(NP // _BR,),
        in_specs=[
            pl.BlockSpec((_BR, 128), lambda i: (i, 0)),
            pl.BlockSpec((128, 128), lambda i: (0, 0)),
        ],
        out_specs=pl.BlockSpec((_BR, 128), lambda i: (i, 0)),
        out_shape=jax.ShapeDtypeStruct((NP, 128), jnp.float32),
    )(x, W1)


def _scale(p, hist, F, slab):
    nb = NP // _BR

    def body(p0, p1, hr, o):
        den = jnp.maximum(hr[...][:, 0:1], 1.0)
        o[...] = (p0[...] + p1[...]) / den

    return pl.pallas_call(
        body,
        grid=(nb,),
        in_specs=[
            pl.BlockSpec((_BR, F), lambda i: (i, 0)),
            pl.BlockSpec((_BR, F), lambda i: (i + nb, 0)),
            pl.BlockSpec((_BR, 128), lambda i, _s=slab * nb: (i + _s, 0)),
        ],
        out_specs=pl.BlockSpec((_BR, F), lambda i: (i, 0)),
        out_shape=jax.ShapeDtypeStruct((NP, F), jnp.float32),
    )(p, p, hist)


def _s2(q, hist, b1, W2p):
    nb = NP // _BR

    def body(q0, q1, hr, br, wr, o):
        den = jnp.maximum(hr[...][:, 0:1], 1.0)
        h = jnp.maximum((q0[...] + q1[...]) / den + br[...], 0.0)
        o[...] = jnp.dot(h, wr[...], preferred_element_type=jnp.float32)

    return pl.pallas_call(
        body,
        grid=(nb,),
        in_specs=[
            pl.BlockSpec((_BR, 128), lambda i: (i, 0)),
            pl.BlockSpec((_BR, 128), lambda i: (i + nb, 0)),
            pl.BlockSpec((_BR, 128), lambda i: (i, 0)),
            pl.BlockSpec((1, 128), lambda i: (0, 0)),
            pl.BlockSpec((128, 128), lambda i: (0, 0)),
        ],
        out_specs=pl.BlockSpec((_BR, 128), lambda i: (i, 0)),
        out_shape=jax.ShapeDtypeStruct((NP, 128), jnp.float32),
    )(q, q, hist, b1, W2p)


def _s4(r, hist, b2):
    nb = NP // _BR

    def body(r0, r1, hr, br, o):
        den = jnp.maximum(hr[...][:, 0:1], 1.0)
        o[...] = (r0[...] + r1[...])[:, :64] / den + br[...]

    return pl.pallas_call(
        body,
        grid=(nb,),
        in_specs=[
            pl.BlockSpec((_BR, 128), lambda i: (i, 0)),
            pl.BlockSpec((_BR, 128), lambda i: (i + nb, 0)),
            pl.BlockSpec((_BR, 128), lambda i: (i, 0)),
            pl.BlockSpec((1, 64), lambda i: (0, 0)),
        ],
        out_specs=pl.BlockSpec((_BR, 64), lambda i: (i, 0)),
        out_shape=jax.ShapeDtypeStruct((NP, 64), jnp.float32),
    )(r, r, hist, b2)


_pass128 = _sc_pass(128)


def kernel(x, edge_index, W1, b1, W2, b2):
    nid = edge_index[0]
    hid = edge_index[1]
    pad = jnp.full((EP - E,), TRASH, dtype=jnp.int32)
    nid_p = jnp.concatenate([nid, pad])
    hid_p = jnp.concatenate([hid, pad])
    x_pad = jnp.zeros((NP, 128), jnp.float32).at[:N].set(x)
    z128 = jnp.zeros((CH, 128), jnp.float32)
    W2p = jnp.zeros((128, 128), jnp.float32).at[:, :64].set(W2)

    hist = _sc_hist(jnp.concatenate([nid_p, hid_p]), z128)
    xt1 = _mm1(x_pad, W1)
    p = _pass128(xt1, nid_p, hid_p, z128)
    he = _scale(p, hist, 128, 1)
    q = _pass128(he, hid_p, nid_p, z128)
    xt2 = _s2(q, hist, b1.reshape(1, -1), W2p)
    u = _pass128(xt2, nid_p, hid_p, z128)
    he2 = _scale(u, hist, 128, 1)
    r = _pass128(he2, hid_p, nid_p, z128)
    out = _s4(r, hist, b2.reshape(1, -1))
    return out[:N]
